# baseline (speedup 1.0000x reference)
import jax
import jax.numpy as jnp
from jax.experimental import pallas as pl


def _copy_body(x_ref, o_ref):
    o_ref[...] = x_ref[...]


def kernel(x, edge_index, W_msg, b_msg, att_msg):
    N, D = x.shape
    src = edge_index[0]
    dst = edge_index[1]
    y = x @ W_msg + b_msg
    a = jnp.sum(y * att_msg.reshape(1, D), axis=-1)
    a = jax.nn.leaky_relu(a, negative_slope=0.2)
    alpha = a[src]
    seg_max = jax.ops.segment_max(alpha, dst, num_segments=N)
    p = jnp.exp(alpha - seg_max[dst])
    seg_sum = jax.ops.segment_sum(p, dst, num_segments=N)
    w = p / (seg_sum[dst] + 1e-16)
    wmsg = y[src] * w[:, None]
    agg = jax.ops.segment_min(wmsg, dst, num_segments=N)
    deg = jax.ops.segment_sum(jnp.ones_like(p), dst, num_segments=N)
    agg = jnp.where((deg > 0)[:, None], agg, 0.0)
    out = agg + x
    return pl.pallas_call(
        _copy_body,
        out_shape=jax.ShapeDtypeStruct(out.shape, out.dtype),
    )(out)


# speedup vs baseline: 4.5782x; 4.5782x over previous
import functools

import jax
import jax.numpy as jnp
from jax import lax
from jax.experimental import pallas as pl
from jax.experimental.pallas import tpu as pltpu
from jax.experimental.pallas import tpu_sc as plsc

N = 10000
NP = 10240
E = 320000
D = 128

NC = 2
NS = 16
NW = NC * NS

EPT = E // NW
CA = 2000
CB = 6400
QS = 2000
FPT = D // NW

_mesh = plsc.VectorSubcoreMesh(core_axis_name="c", subcore_axis_name="s")
_sc_params = pltpu.CompilerParams(needs_layout_passes=False)


def _dense_body(x_ref, W_ref, b_ref, att_ref, yT_ref, a_ref, m_ref):
    j = pl.program_id(0)
    y = jnp.dot(x_ref[...], W_ref[...], preferred_element_type=jnp.float32)
    y = y + b_ref[...]
    yT = y.T
    yT_ref[...] = yT
    av = jnp.sum(yT * att_ref[...], axis=0, keepdims=True)
    av = jnp.where(av > 0, av, 0.2 * av)
    a_ref[...] = av
    bm = jnp.max(av)

    @pl.when(j == 0)
    def _():
        m_ref[0, 0] = bm

    @pl.when(j > 0)
    def _():
        m_ref[0, 0] = jnp.maximum(m_ref[0, 0], bm)


def _dense(xp, W, b, att):
    nb = NP // 1024
    return pl.pallas_call(
        _dense_body,
        grid=(nb,),
        in_specs=[
            pl.BlockSpec((1024, D), lambda j: (j, 0)),
            pl.BlockSpec((D, D), lambda j: (0, 0)),
            pl.BlockSpec((1, D), lambda j: (0, 0)),
            pl.BlockSpec((D, 1), lambda j: (0, 0)),
        ],
        out_specs=[
            pl.BlockSpec((D, 1024), lambda j: (0, j)),
            pl.BlockSpec((1, 1024), lambda j: (0, j)),
            pl.BlockSpec(memory_space=pltpu.SMEM),
        ],
        out_shape=[
            jax.ShapeDtypeStruct((D, NP), jnp.float32),
            jax.ShapeDtypeStruct((1, NP), jnp.float32),
            jax.ShapeDtypeStruct((1, 1), jnp.float32),
        ],
    )(xp, W, b, att)


def _segsum_body(src_h, dst_h, a2_h, zero_h, s2_h, a2_v, srcb, dstb, pb, s_sh):
    cid = lax.axis_index("c")
    sid = lax.axis_index("s")
    wid = cid * NS + sid
    pltpu.sync_copy(a2_h, a2_v)

    @pl.when(sid == 0)
    def _():
        pltpu.sync_copy(zero_h, s_sh)

    plsc.subcore_barrier()

    base = wid * EPT

    def chunk(ci, carry):
        off = base + ci * CA
        pltpu.sync_copy(src_h.at[pl.ds(off, CA)], srcb)
        pltpu.sync_copy(dst_h.at[pl.ds(off, CA)], dstb)

        def vec(i, c2):
            s = srcb[pl.ds(i * 16, 16)]
            av = plsc.load_gather(a2_v, [s])
            pb[pl.ds(i * 16, 16)] = jnp.exp(av)
            return c2

        lax.fori_loop(0, CA // 16, vec, 0)
        pltpu.sync_copy(pb, s_sh.at[dstb], add=True)
        return carry

    lax.fori_loop(0, EPT // CA, chunk, 0)
    plsc.subcore_barrier()

    @pl.when(sid == 0)
    def _():
        pltpu.sync_copy(s_sh, s2_h.at[pl.ds(cid * NP, NP)])


_segsum = functools.partial(
    pl.kernel,
    out_type=jax.ShapeDtypeStruct((2 * NP,), jnp.float32),
    mesh=_mesh,
    scratch_types=[
        pltpu.VMEM((N,), jnp.float32),
        pltpu.VMEM((CA,), jnp.int32),
        pltpu.VMEM((CA,), jnp.int32),
        pltpu.VMEM((CA,), jnp.float32),
        pltpu.VMEM_SHARED((NP,), jnp.float32),
    ],
    compiler_params=_sc_params,
)(_segsum_body)


def _minagg_body(src_h, dst_h, a2_h, yT_h, s2_h, outT_h,
                 a2_v, ybuf, outb, sinv, srcb, dstb, t0, t1):
    cid = lax.axis_index("c")
    sid = lax.axis_index("s")
    wid = cid * NS + sid

    pltpu.sync_copy(a2_h, a2_v)
    pltpu.sync_copy(yT_h.at[pl.ds(wid * FPT * NP, FPT * NP)], ybuf)

    def stage(k, carry):
        pltpu.sync_copy(s2_h.at[pl.ds(k * QS, QS)], t0)
        pltpu.sync_copy(s2_h.at[pl.ds(NP + k * QS, QS)], t1)

        def vec(i, c2):
            s = t0[pl.ds(i * 16, 16)] + t1[pl.ds(i * 16, 16)]
            sinv[pl.ds(k * QS + i * 16, 16)] = 1.0 / (s + 1e-16)
            return c2

        lax.fori_loop(0, QS // 16, vec, 0)
        return carry

    lax.fori_loop(0, N // QS, stage, 0)

    inf16 = jnp.full((16,), jnp.inf, jnp.float32)

    def init(i, carry):
        outb[pl.ds(i * 16, 16)] = inf16
        return carry

    lax.fori_loop(0, (FPT * NP) // 16, init, 0)

    def chunk(ci, carry):
        off = ci * CB
        pltpu.sync_copy(src_h.at[pl.ds(off, CB)], srcb)
        pltpu.sync_copy(dst_h.at[pl.ds(off, CB)], dstb)

        def vec(i, c2):
            s = srcb[pl.ds(i * 16, 16)]
            d = dstb[pl.ds(i * 16, 16)]
            av = plsc.load_gather(a2_v, [s])
            p = jnp.exp(av)
            iv = plsc.load_gather(sinv, [d])
            w = p * iv
            v0 = w * plsc.load_gather(ybuf, [s])
            v1 = w * plsc.load_gather(ybuf, [s + NP])
            v2 = w * plsc.load_gather(ybuf, [s + 2 * NP])
            v3 = w * plsc.load_gather(ybuf, [s + 3 * NP])
            cnt, _ = plsc.scan_count(d)
            rmin = jnp.min(cnt)
            rmax = jnp.max(cnt)

            def rmw(r, c3):
                m = cnt == r
                c0 = plsc.load_gather(outb, [d], mask=m)
                plsc.store_scatter(outb, [d], jnp.minimum(c0, v0), mask=m)
                c1 = plsc.load_gather(outb, [d + NP], mask=m)
                plsc.store_scatter(outb, [d + NP], jnp.minimum(c1, v1), mask=m)
                c2_ = plsc.load_gather(outb, [d + 2 * NP], mask=m)
                plsc.store_scatter(outb, [d + 2 * NP], jnp.minimum(c2_, v2), mask=m)
                c3_ = plsc.load_gather(outb, [d + 3 * NP], mask=m)
                plsc.store_scatter(outb, [d + 3 * NP], jnp.minimum(c3_, v3), mask=m)
                return c3

            lax.fori_loop(rmin, rmax + 1, rmw, 0)
            return c2

        lax.fori_loop(0, CB // 16, vec, 0)
        return carry

    lax.fori_loop(0, E // CB, chunk, 0)
    pltpu.sync_copy(outb, outT_h.at[pl.ds(wid * FPT * NP, FPT * NP)])


_minagg = functools.partial(
    pl.kernel,
    out_type=jax.ShapeDtypeStruct((D * NP,), jnp.float32),
    mesh=_mesh,
    scratch_types=[
        pltpu.VMEM((N,), jnp.float32),
        pltpu.VMEM((FPT * NP,), jnp.float32),
        pltpu.VMEM((FPT * NP,), jnp.float32),
        pltpu.VMEM((N,), jnp.float32),
        pltpu.VMEM((CB,), jnp.int32),
        pltpu.VMEM((CB,), jnp.int32),
        pltpu.VMEM((QS,), jnp.float32),
        pltpu.VMEM((QS,), jnp.float32),
    ],
    compiler_params=_sc_params,
)(_minagg_body)


def _epi_body(outT_ref, s2_ref, x_ref, o_ref):
    agg = outT_ref[...]
    s = s2_ref[0:1, :] + s2_ref[1:2, :]
    agg = jnp.where(s > 0, agg, 0.0)
    o_ref[...] = agg.T + x_ref[...]


def _epilogue(outT, s2, xp):
    nb = NP // 1024
    return pl.pallas_call(
        _epi_body,
        grid=(nb,),
        in_specs=[
            pl.BlockSpec((D, 1024), lambda j: (0, j)),
            pl.BlockSpec((2, 1024), lambda j: (0, j)),
            pl.BlockSpec((1024, D), lambda j: (j, 0)),
        ],
        out_specs=pl.BlockSpec((1024, D), lambda j: (j, 0)),
        out_shape=jax.ShapeDtypeStruct((NP, D), jnp.float32),
    )(outT, s2, xp)


def kernel(x, edge_index, W_msg, b_msg, att_msg):
    src = edge_index[0]
    dst = edge_index[1]
    xp = jnp.pad(x, ((0, NP - N), (0, 0)))
    yT, a, m = _dense(xp, W_msg, b_msg.reshape(1, D), att_msg.reshape(D, 1))
    a2 = a[0, :N] - m[0, 0]
    zeros = jnp.zeros((NP,), jnp.float32)
    s2 = _segsum(src, dst, a2, zeros)
    outT = _minagg(src, dst, a2, yT.reshape(D * NP), s2)
    out = _epilogue(outT.reshape(D, NP), s2.reshape(2, NP), xp)
    return out[:N]


# speedup vs baseline: 5.4430x; 1.1889x over previous
import functools

import jax
import jax.numpy as jnp
from jax import lax
from jax.experimental import pallas as pl
from jax.experimental.pallas import tpu as pltpu
from jax.experimental.pallas import tpu_sc as plsc

N = 10000
NP = 10240
E = 320000
D = 128

NC = 2
NS = 16
NW = NC * NS

EPT = E // NW
CA = 2000
CB = 4000
QS = 2000
FPT = D // NW

_mesh = plsc.VectorSubcoreMesh(core_axis_name="c", subcore_axis_name="s")
_sc_params = pltpu.CompilerParams(needs_layout_passes=False)


def _dense_body(x_ref, W_ref, b_ref, att_ref, yT_ref, a_ref, m_ref):
    j = pl.program_id(0)
    y = jnp.dot(x_ref[...], W_ref[...], preferred_element_type=jnp.float32)
    y = y + b_ref[...]
    yT = y.T
    yT_ref[...] = yT
    av = jnp.sum(yT * att_ref[...], axis=0, keepdims=True)
    av = jnp.where(av > 0, av, 0.2 * av)
    a_ref[...] = av
    bm = jnp.max(av)

    @pl.when(j == 0)
    def _():
        m_ref[0, 0] = bm

    @pl.when(j > 0)
    def _():
        m_ref[0, 0] = jnp.maximum(m_ref[0, 0], bm)


def _dense(xp, W, b, att):
    nb = NP // 1024
    return pl.pallas_call(
        _dense_body,
        grid=(nb,),
        in_specs=[
            pl.BlockSpec((1024, D), lambda j: (j, 0)),
            pl.BlockSpec((D, D), lambda j: (0, 0)),
            pl.BlockSpec((1, D), lambda j: (0, 0)),
            pl.BlockSpec((D, 1), lambda j: (0, 0)),
        ],
        out_specs=[
            pl.BlockSpec((D, 1024), lambda j: (0, j)),
            pl.BlockSpec((1, 1024), lambda j: (0, j)),
            pl.BlockSpec(memory_space=pltpu.SMEM),
        ],
        out_shape=[
            jax.ShapeDtypeStruct((D, NP), jnp.float32),
            jax.ShapeDtypeStruct((1, NP), jnp.float32),
            jax.ShapeDtypeStruct((1, 1), jnp.float32),
        ],
    )(xp, W, b, att)


def _segsum_body(src_h, dst_h, a2_h, zero_h, s2_h, a2_v, srcb, dstb, pb, s_sh):
    cid = lax.axis_index("c")
    sid = lax.axis_index("s")
    wid = cid * NS + sid
    pltpu.sync_copy(a2_h, a2_v)

    @pl.when(sid == 0)
    def _():
        pltpu.sync_copy(zero_h, s_sh)

    plsc.subcore_barrier()

    base = wid * EPT

    def chunk(ci, carry):
        off = base + ci * CA
        pltpu.sync_copy(src_h.at[pl.ds(off, CA)], srcb)
        pltpu.sync_copy(dst_h.at[pl.ds(off, CA)], dstb)

        def vec(i, c2):
            s = srcb[pl.ds(i * 16, 16)]
            av = plsc.load_gather(a2_v, [s])
            pb[pl.ds(i * 16, 16)] = jnp.exp(av)
            return c2

        lax.fori_loop(0, CA // 16, vec, 0)
        pltpu.sync_copy(pb, s_sh.at[dstb], add=True)
        return carry

    lax.fori_loop(0, EPT // CA, chunk, 0)
    plsc.subcore_barrier()

    @pl.when(sid == 0)
    def _():
        pltpu.sync_copy(s_sh, s2_h.at[pl.ds(cid * NP, NP)])


_segsum = functools.partial(
    pl.kernel,
    out_type=jax.ShapeDtypeStruct((2 * NP,), jnp.float32),
    mesh=_mesh,
    scratch_types=[
        pltpu.VMEM((N,), jnp.float32),
        pltpu.VMEM((CA,), jnp.int32),
        pltpu.VMEM((CA,), jnp.int32),
        pltpu.VMEM((CA,), jnp.float32),
        pltpu.VMEM_SHARED((NP,), jnp.float32),
    ],
    compiler_params=_sc_params,
)(_segsum_body)


def _edgew_body(src_h, dst_h, a2_h, s2_h, w_h, a2_v, sinv, t0, t1, srcb, dstb, wb):
    cid = lax.axis_index("c")
    sid = lax.axis_index("s")
    wid = cid * NS + sid
    pltpu.sync_copy(a2_h, a2_v)

    def stage(k, carry):
        pltpu.sync_copy(s2_h.at[pl.ds(k * QS, QS)], t0)
        pltpu.sync_copy(s2_h.at[pl.ds(NP + k * QS, QS)], t1)

        def vec(i, c2):
            s = t0[pl.ds(i * 16, 16)] + t1[pl.ds(i * 16, 16)]
            sinv[pl.ds(k * QS + i * 16, 16)] = 1.0 / (s + 1e-16)
            return c2

        lax.fori_loop(0, QS // 16, vec, 0)
        return carry

    lax.fori_loop(0, N // QS, stage, 0)

    base = wid * EPT

    def chunk(ci, carry):
        off = base + ci * CA
        pltpu.sync_copy(src_h.at[pl.ds(off, CA)], srcb)
        pltpu.sync_copy(dst_h.at[pl.ds(off, CA)], dstb)

        def vec(i, c2):
            s = srcb[pl.ds(i * 16, 16)]
            d = dstb[pl.ds(i * 16, 16)]
            p = jnp.exp(plsc.load_gather(a2_v, [s]))
            iv = plsc.load_gather(sinv, [d])
            wb[pl.ds(i * 16, 16)] = p * iv
            return c2

        lax.fori_loop(0, CA // 16, vec, 0)
        pltpu.sync_copy(wb, w_h.at[pl.ds(off, CA)])
        return carry

    lax.fori_loop(0, EPT // CA, chunk, 0)


_edgew = functools.partial(
    pl.kernel,
    out_type=jax.ShapeDtypeStruct((E,), jnp.float32),
    mesh=_mesh,
    scratch_types=[
        pltpu.VMEM((N,), jnp.float32),
        pltpu.VMEM((N,), jnp.float32),
        pltpu.VMEM((QS,), jnp.float32),
        pltpu.VMEM((QS,), jnp.float32),
        pltpu.VMEM((CA,), jnp.int32),
        pltpu.VMEM((CA,), jnp.int32),
        pltpu.VMEM((CA,), jnp.float32),
    ],
    compiler_params=_sc_params,
)(_edgew_body)


_IOTA16 = None


def _minagg_body(src_h, dst_h, w_h, yT_h, outT_h,
                 ybuf, outb, dupchk,
                 sb0, db0, wb0, sb1, db1, wb1, sem0, sem1):
    cid = lax.axis_index("c")
    sid = lax.axis_index("s")
    wid = cid * NS + sid

    pltpu.sync_copy(yT_h.at[pl.ds(wid * FPT * NP, FPT * NP)], ybuf)

    inf16 = jnp.full((16,), jnp.inf, jnp.float32)

    def init(i, carry):
        outb[pl.ds(i * 16, 16)] = inf16
        return carry

    lax.fori_loop(0, (FPT * NP) // 16, init, 0)

    sbufs = (sb0, sb1)
    dbufs = (db0, db1)
    wbufs = (wb0, wb1)
    sems = (sem0, sem1)
    nchunks = E // CB
    nv = CB // 16
    iota16 = lax.iota(jnp.int32, 16)

    def start(ci, b):
        off = ci * CB
        pltpu.async_copy(src_h.at[pl.ds(off, CB)], sbufs[b], sems[b])
        pltpu.async_copy(dst_h.at[pl.ds(off, CB)], dbufs[b], sems[b])
        pltpu.async_copy(w_h.at[pl.ds(off, CB)], wbufs[b], sems[b])

    def wait(ci, b):
        off = ci * CB
        pltpu.make_async_copy(src_h.at[pl.ds(off, CB)], sbufs[b], sems[b]).wait()
        pltpu.make_async_copy(dst_h.at[pl.ds(off, CB)], dbufs[b], sems[b]).wait()
        pltpu.make_async_copy(w_h.at[pl.ds(off, CB)], wbufs[b], sems[b]).wait()

    for b in (0, 1):
        start(b, b)

    def process(b):
        sb, db, wb = sbufs[b], dbufs[b], wbufs[b]

        def vec(i, carry):
            s = sb[pl.ds(i * 16, 16)]
            d = db[pl.ds(i * 16, 16)]
            w = wb[pl.ds(i * 16, 16)]
            v0 = w * plsc.load_gather(ybuf, [s])
            v1 = w * plsc.load_gather(ybuf, [s + NP])
            v2 = w * plsc.load_gather(ybuf, [s + 2 * NP])
            v3 = w * plsc.load_gather(ybuf, [s + 3 * NP])
            plsc.store_scatter(dupchk, [d], iota16)
            rd = plsc.load_gather(dupchk, [d])
            ndup = plsc.all_reduce_population_count(rd != iota16)[0]

            def fast(_):
                c0 = plsc.load_gather(outb, [d])
                plsc.store_scatter(outb, [d], jnp.minimum(c0, v0))
                c1 = plsc.load_gather(outb, [d + NP])
                plsc.store_scatter(outb, [d + NP], jnp.minimum(c1, v1))
                c2 = plsc.load_gather(outb, [d + 2 * NP])
                plsc.store_scatter(outb, [d + 2 * NP], jnp.minimum(c2, v2))
                c3 = plsc.load_gather(outb, [d + 3 * NP])
                plsc.store_scatter(outb, [d + 3 * NP], jnp.minimum(c3, v3))
                return 0

            def slow(_):
                def rnd(_r, m):
                    lost = jnp.zeros((16,), jnp.bool_)
                    for dd, vv in ((d, v0), (d + NP, v1),
                                   (d + 2 * NP, v2), (d + 3 * NP, v3)):
                        c = plsc.load_gather(outb, [dd], mask=m)
                        nvv = jnp.minimum(c, vv)
                        plsc.store_scatter(outb, [dd], nvv, mask=m)
                        chk = plsc.load_gather(outb, [dd], mask=m)
                        lost = jnp.logical_or(lost, jnp.logical_and(m, chk > nvv))
                    return lost

                lax.fori_loop(0, 16, rnd, jnp.full((16,), True, jnp.bool_))
                return 0

            lax.cond(ndup == 0, fast, slow, 0)
            return carry

        lax.fori_loop(0, nv, vec, 0)

    def pair(cj, carry):
        for b in (0, 1):
            ci = cj * 2 + b
            wait(ci, b)
            process(b)
            nci = ci + 2

            @pl.when(nci < nchunks)
            def _():
                start(nci, b)

        return carry

    lax.fori_loop(0, nchunks // 2, pair, 0)
    pltpu.sync_copy(outb, outT_h.at[pl.ds(wid * FPT * NP, FPT * NP)])


_minagg = functools.partial(
    pl.kernel,
    out_type=jax.ShapeDtypeStruct((D * NP,), jnp.float32),
    mesh=_mesh,
    scratch_types=[
        pltpu.VMEM((FPT * NP,), jnp.float32),
        pltpu.VMEM((FPT * NP,), jnp.float32),
        pltpu.VMEM((N,), jnp.int32),
        pltpu.VMEM((CB,), jnp.int32),
        pltpu.VMEM((CB,), jnp.int32),
        pltpu.VMEM((CB,), jnp.float32),
        pltpu.VMEM((CB,), jnp.int32),
        pltpu.VMEM((CB,), jnp.int32),
        pltpu.VMEM((CB,), jnp.float32),
        pltpu.SemaphoreType.DMA,
        pltpu.SemaphoreType.DMA,
    ],
    compiler_params=_sc_params,
)(_minagg_body)


def _epi_body(outT_ref, s2_ref, x_ref, o_ref):
    agg = outT_ref[...]
    s = s2_ref[0:1, :] + s2_ref[1:2, :]
    agg = jnp.where(s > 0, agg, 0.0)
    o_ref[...] = agg.T + x_ref[...]


def _epilogue(outT, s2, xp):
    nb = NP // 1024
    return pl.pallas_call(
        _epi_body,
        grid=(nb,),
        in_specs=[
            pl.BlockSpec((D, 1024), lambda j: (0, j)),
            pl.BlockSpec((2, 1024), lambda j: (0, j)),
            pl.BlockSpec((1024, D), lambda j: (j, 0)),
        ],
        out_specs=pl.BlockSpec((1024, D), lambda j: (j, 0)),
        out_shape=jax.ShapeDtypeStruct((NP, D), jnp.float32),
    )(outT, s2, xp)


def kernel(x, edge_index, W_msg, b_msg, att_msg):
    src = edge_index[0]
    dst = edge_index[1]
    xp = jnp.pad(x, ((0, NP - N), (0, 0)))
    yT, a, m = _dense(xp, W_msg, b_msg.reshape(1, D), att_msg.reshape(D, 1))
    a2 = a[0, :N] - m[0, 0]
    zeros = jnp.zeros((NP,), jnp.float32)
    s2 = _segsum(src, dst, a2, zeros)
    w = _edgew(src, dst, a2, s2)
    outT = _minagg(src, dst, w, yT.reshape(D * NP))
    out = _epilogue(outT.reshape(D, NP), s2.reshape(2, NP), xp)
    return out[:N]


# speedup vs baseline: 7.9402x; 1.4588x over previous
import functools

import jax
import jax.numpy as jnp
from jax import lax
from jax.experimental import pallas as pl
from jax.experimental.pallas import tpu as pltpu
from jax.experimental.pallas import tpu_sc as plsc

N = 10000
NP = 10240
E = 320000
D = 128

NC = 2
NS = 16
NW = NC * NS

EPT = E // NW
CA = 2000
CB = 4000
QS = 2000
FPT = D // NW

_mesh = plsc.VectorSubcoreMesh(core_axis_name="c", subcore_axis_name="s")
_sc_params = pltpu.CompilerParams(needs_layout_passes=False)


def _dense_body(x_ref, W_ref, b_ref, att_ref, yT_ref, a_ref, m_ref):
    j = pl.program_id(0)
    y = jnp.dot(x_ref[...], W_ref[...], preferred_element_type=jnp.float32)
    y = y + b_ref[...]
    yT = y.T
    yT_ref[...] = yT
    av = jnp.sum(yT * att_ref[...], axis=0, keepdims=True)
    av = jnp.where(av > 0, av, 0.2 * av)
    a_ref[...] = av
    bm = jnp.max(av)

    @pl.when(j == 0)
    def _():
        m_ref[0, 0] = bm

    @pl.when(j > 0)
    def _():
        m_ref[0, 0] = jnp.maximum(m_ref[0, 0], bm)


def _dense(xp, W, b, att):
    nb = NP // 1024
    return pl.pallas_call(
        _dense_body,
        grid=(nb,),
        in_specs=[
            pl.BlockSpec((1024, D), lambda j: (j, 0)),
            pl.BlockSpec((D, D), lambda j: (0, 0)),
            pl.BlockSpec((1, D), lambda j: (0, 0)),
            pl.BlockSpec((D, 1), lambda j: (0, 0)),
        ],
        out_specs=[
            pl.BlockSpec((D, 1024), lambda j: (0, j)),
            pl.BlockSpec((1, 1024), lambda j: (0, j)),
            pl.BlockSpec(memory_space=pltpu.SMEM),
        ],
        out_shape=[
            jax.ShapeDtypeStruct((D, NP), jnp.float32),
            jax.ShapeDtypeStruct((1, NP), jnp.float32),
            jax.ShapeDtypeStruct((1, 1), jnp.float32),
        ],
    )(xp, W, b, att)


def _segsum_body(src_h, dst_h, a2_h, zero_h, s2_h, a2_v, srcb, dstb, pb, s_sh):
    cid = lax.axis_index("c")
    sid = lax.axis_index("s")
    wid = cid * NS + sid
    pltpu.sync_copy(a2_h, a2_v)

    @pl.when(sid == 0)
    def _():
        pltpu.sync_copy(zero_h, s_sh)

    plsc.subcore_barrier()

    base = wid * EPT

    def chunk(ci, carry):
        off = base + ci * CA
        pltpu.sync_copy(src_h.at[pl.ds(off, CA)], srcb)
        pltpu.sync_copy(dst_h.at[pl.ds(off, CA)], dstb)

        def vec(i, c2):
            s = srcb[pl.ds(i * 16, 16)]
            av = plsc.load_gather(a2_v, [s])
            pb[pl.ds(i * 16, 16)] = jnp.exp(av)
            return c2

        lax.fori_loop(0, CA // 16, vec, 0)
        pltpu.sync_copy(pb, s_sh.at[dstb], add=True)
        return carry

    lax.fori_loop(0, EPT // CA, chunk, 0)
    plsc.subcore_barrier()

    @pl.when(sid == 0)
    def _():
        pltpu.sync_copy(s_sh, s2_h.at[pl.ds(cid * NP, NP)])


_segsum = functools.partial(
    pl.kernel,
    out_type=jax.ShapeDtypeStruct((2 * NP,), jnp.float32),
    mesh=_mesh,
    scratch_types=[
        pltpu.VMEM((N,), jnp.float32),
        pltpu.VMEM((CA,), jnp.int32),
        pltpu.VMEM((CA,), jnp.int32),
        pltpu.VMEM((CA,), jnp.float32),
        pltpu.VMEM_SHARED((NP,), jnp.float32),
    ],
    compiler_params=_sc_params,
)(_segsum_body)


def _edgew_body(src_h, dst_h, a2_h, s2_h, w_h, a2_v, sinv, t0, t1, srcb, dstb, wb):
    cid = lax.axis_index("c")
    sid = lax.axis_index("s")
    wid = cid * NS + sid
    pltpu.sync_copy(a2_h, a2_v)

    def stage(k, carry):
        pltpu.sync_copy(s2_h.at[pl.ds(k * QS, QS)], t0)
        pltpu.sync_copy(s2_h.at[pl.ds(NP + k * QS, QS)], t1)

        def vec(i, c2):
            s = t0[pl.ds(i * 16, 16)] + t1[pl.ds(i * 16, 16)]
            sinv[pl.ds(k * QS + i * 16, 16)] = 1.0 / (s + 1e-16)
            return c2

        lax.fori_loop(0, QS // 16, vec, 0)
        return carry

    lax.fori_loop(0, N // QS, stage, 0)

    base = wid * EPT

    def chunk(ci, carry):
        off = base + ci * CA
        pltpu.sync_copy(src_h.at[pl.ds(off, CA)], srcb)
        pltpu.sync_copy(dst_h.at[pl.ds(off, CA)], dstb)

        def vec(i, c2):
            s = srcb[pl.ds(i * 16, 16)]
            d = dstb[pl.ds(i * 16, 16)]
            p = jnp.exp(plsc.load_gather(a2_v, [s]))
            iv = plsc.load_gather(sinv, [d])
            wb[pl.ds(i * 16, 16)] = p * iv
            return c2

        lax.fori_loop(0, CA // 16, vec, 0)
        pltpu.sync_copy(wb, w_h.at[pl.ds(off, CA)])
        return carry

    lax.fori_loop(0, EPT // CA, chunk, 0)


_edgew = functools.partial(
    pl.kernel,
    out_type=jax.ShapeDtypeStruct((E,), jnp.float32),
    mesh=_mesh,
    scratch_types=[
        pltpu.VMEM((N,), jnp.float32),
        pltpu.VMEM((N,), jnp.float32),
        pltpu.VMEM((QS,), jnp.float32),
        pltpu.VMEM((QS,), jnp.float32),
        pltpu.VMEM((CA,), jnp.int32),
        pltpu.VMEM((CA,), jnp.int32),
        pltpu.VMEM((CA,), jnp.float32),
    ],
    compiler_params=_sc_params,
)(_edgew_body)


def _minagg_body(src_h, dst_h, w_h, ypk_h, outT_h,
                 ybuf, outb, dupchk,
                 sb0, db0, wb0, sb1, db1, wb1, sem0, sem1):
    cid = lax.axis_index("c")
    sid = lax.axis_index("s")
    wid = cid * NS + sid

    pltpu.sync_copy(ypk_h.at[pl.ds(wid * 2 * NP, 2 * NP)], ybuf)

    inf16 = jnp.full((16,), jnp.inf, jnp.float32)

    def init(i, carry):
        outb[pl.ds(i * 16, 16)] = inf16
        return carry

    lax.fori_loop(0, (FPT * NP) // 16, init, 0)

    sbufs = (sb0, sb1)
    dbufs = (db0, db1)
    wbufs = (wb0, wb1)
    sems = (sem0, sem1)
    nchunks = E // CB
    nv = CB // 16
    iota16 = lax.iota(jnp.int32, 16)

    def start(ci, b):
        off = ci * CB
        pltpu.async_copy(src_h.at[pl.ds(off, CB)], sbufs[b], sems[b])
        pltpu.async_copy(dst_h.at[pl.ds(off, CB)], dbufs[b], sems[b])
        pltpu.async_copy(w_h.at[pl.ds(off, CB)], wbufs[b], sems[b])

    def wait(ci, b):
        off = ci * CB
        pltpu.make_async_copy(src_h.at[pl.ds(off, CB)], sbufs[b], sems[b]).wait()
        pltpu.make_async_copy(dst_h.at[pl.ds(off, CB)], dbufs[b], sems[b]).wait()
        pltpu.make_async_copy(w_h.at[pl.ds(off, CB)], wbufs[b], sems[b]).wait()

    for b in (0, 1):
        start(b, b)

    iota16b = iota16 + 16
    m16 = jnp.int32(-65536)

    def load_group(sb, db, wb, i):
        s = sb[pl.ds(i * 16, 16)]
        d = db[pl.ds(i * 16, 16)]
        w = wb[pl.ds(i * 16, 16)]
        ya = plsc.load_gather(ybuf, [s])
        yb = plsc.load_gather(ybuf, [s + NP])
        f0 = lax.bitcast_convert_type(lax.shift_left(ya, 16), jnp.float32)
        f1 = lax.bitcast_convert_type(lax.bitwise_and(ya, m16), jnp.float32)
        f2 = lax.bitcast_convert_type(lax.shift_left(yb, 16), jnp.float32)
        f3 = lax.bitcast_convert_type(lax.bitwise_and(yb, m16), jnp.float32)
        return d, (w * f0, w * f1, w * f2, w * f3)

    def rmw_fast(d, vs):
        for k in range(4):
            dd = d + k * NP
            c = plsc.load_gather(outb, [dd])
            plsc.store_scatter(outb, [dd], jnp.minimum(c, vs[k]))

    def rmw_retry(d, vs):
        def cond(m):
            return plsc.all_reduce_population_count(m)[0] > 0

        def body(m):
            lost = jnp.zeros((16,), jnp.bool_)
            for k in range(4):
                dd = d + k * NP
                c = plsc.load_gather(outb, [dd], mask=m)
                nvv = jnp.minimum(c, vs[k])
                plsc.store_scatter(outb, [dd], nvv, mask=m)
                chk = plsc.load_gather(outb, [dd], mask=m)
                lost = jnp.logical_or(lost, jnp.logical_and(m, chk > nvv))
            return lost

        lax.while_loop(cond, body, jnp.full((16,), True, jnp.bool_))

    def process(b):
        sb, db, wb = sbufs[b], dbufs[b], wbufs[b]

        def vec(i, carry):
            dA, vA = load_group(sb, db, wb, 2 * i)
            dB, vB = load_group(sb, db, wb, 2 * i + 1)
            plsc.store_scatter(dupchk, [dA], iota16)
            plsc.store_scatter(dupchk, [dB], iota16b)
            rdA = plsc.load_gather(dupchk, [dA])
            rdB = plsc.load_gather(dupchk, [dB])
            bad = jnp.logical_or(rdA != iota16, rdB != iota16b)
            nbad = plsc.all_reduce_population_count(bad)[0]

            def fast(_):
                rmw_fast(dA, vA)
                rmw_fast(dB, vB)
                return 0

            def slow(_):
                rmw_retry(dA, vA)
                rmw_retry(dB, vB)
                return 0

            lax.cond(nbad == 0, fast, slow, 0)
            return carry

        lax.fori_loop(0, nv // 2, vec, 0)

    def pair(cj, carry):
        for b in (0, 1):
            ci = cj * 2 + b
            wait(ci, b)
            process(b)
            nci = ci + 2

            @pl.when(nci < nchunks)
            def _():
                start(nci, b)

        return carry

    lax.fori_loop(0, nchunks // 2, pair, 0)
    pltpu.sync_copy(outb, outT_h.at[pl.ds(wid * FPT * NP, FPT * NP)])


_minagg = functools.partial(
    pl.kernel,
    out_type=jax.ShapeDtypeStruct((D * NP,), jnp.float32),
    mesh=_mesh,
    scratch_types=[
        pltpu.VMEM((2 * NP,), jnp.int32),
        pltpu.VMEM((FPT * NP,), jnp.float32),
        pltpu.VMEM((N,), jnp.int32),
        pltpu.VMEM((CB,), jnp.int32),
        pltpu.VMEM((CB,), jnp.int32),
        pltpu.VMEM((CB,), jnp.float32),
        pltpu.VMEM((CB,), jnp.int32),
        pltpu.VMEM((CB,), jnp.int32),
        pltpu.VMEM((CB,), jnp.float32),
        pltpu.SemaphoreType.DMA,
        pltpu.SemaphoreType.DMA,
    ],
    compiler_params=_sc_params,
)(_minagg_body)


def _epi_body(outT_ref, s2_ref, x_ref, o_ref):
    agg = outT_ref[...]
    s = s2_ref[0:1, :] + s2_ref[1:2, :]
    agg = jnp.where(s > 0, agg, 0.0)
    o_ref[...] = agg.T + x_ref[...]


def _epilogue(outT, s2, xp):
    nb = NP // 1024
    return pl.pallas_call(
        _epi_body,
        grid=(nb,),
        in_specs=[
            pl.BlockSpec((D, 1024), lambda j: (0, j)),
            pl.BlockSpec((2, 1024), lambda j: (0, j)),
            pl.BlockSpec((1024, D), lambda j: (j, 0)),
        ],
        out_specs=pl.BlockSpec((1024, D), lambda j: (j, 0)),
        out_shape=jax.ShapeDtypeStruct((NP, D), jnp.float32),
    )(outT, s2, xp)


def kernel(x, edge_index, W_msg, b_msg, att_msg):
    src = edge_index[0]
    dst = edge_index[1]
    xp = jnp.pad(x, ((0, NP - N), (0, 0)))
    yT, a, m = _dense(xp, W_msg, b_msg.reshape(1, D), att_msg.reshape(D, 1))
    a2 = a[0, :N] - m[0, 0]
    zeros = jnp.zeros((NP,), jnp.float32)
    s2 = _segsum(src, dst, a2, zeros)
    w = _edgew(src, dst, a2, s2)
    yb16 = lax.bitcast_convert_type(yT.astype(jnp.bfloat16), jnp.uint16)
    ypk = (yb16[0::2, :].astype(jnp.uint32)
           | (yb16[1::2, :].astype(jnp.uint32) << 16))
    ypk = lax.bitcast_convert_type(ypk, jnp.int32).reshape(D // 2 * NP)
    outT = _minagg(src, dst, w, ypk)
    out = _epilogue(outT.reshape(D, NP), s2.reshape(2, NP), xp)
    return out[:N]


# speedup vs baseline: 12.8013x; 1.6122x over previous
import functools

import jax
import jax.numpy as jnp
from jax import lax
from jax.experimental import pallas as pl
from jax.experimental.pallas import tpu as pltpu
from jax.experimental.pallas import tpu_sc as plsc

N = 10000
NP = 10240
E = 320000
D = 128

NC = 2
NS = 16
NW = NC * NS

EPT = E // NW
CA = 2000
CB = 4000
QS = 2000
FPT = D // NW

_mesh = plsc.VectorSubcoreMesh(core_axis_name="c", subcore_axis_name="s")
_sc_params = pltpu.CompilerParams(needs_layout_passes=False)


def _dense_body(x_ref, W_ref, b_ref, att_ref, yT_ref, a_ref, m_ref):
    j = pl.program_id(0)
    y = jnp.dot(x_ref[...], W_ref[...], preferred_element_type=jnp.float32)
    y = y + b_ref[...]
    yT = y.T
    yT_ref[...] = yT
    av = jnp.sum(yT * att_ref[...], axis=0, keepdims=True)
    av = jnp.where(av > 0, av, 0.2 * av)
    a_ref[...] = av
    bm = jnp.max(av)

    @pl.when(j == 0)
    def _():
        m_ref[0, 0] = bm

    @pl.when(j > 0)
    def _():
        m_ref[0, 0] = jnp.maximum(m_ref[0, 0], bm)


def _dense(xp, W, b, att):
    nb = NP // 1024
    return pl.pallas_call(
        _dense_body,
        grid=(nb,),
        in_specs=[
            pl.BlockSpec((1024, D), lambda j: (j, 0)),
            pl.BlockSpec((D, D), lambda j: (0, 0)),
            pl.BlockSpec((1, D), lambda j: (0, 0)),
            pl.BlockSpec((D, 1), lambda j: (0, 0)),
        ],
        out_specs=[
            pl.BlockSpec((D, 1024), lambda j: (0, j)),
            pl.BlockSpec((1, 1024), lambda j: (0, j)),
            pl.BlockSpec(memory_space=pltpu.SMEM),
        ],
        out_shape=[
            jax.ShapeDtypeStruct((D, NP), jnp.float32),
            jax.ShapeDtypeStruct((1, NP), jnp.float32),
            jax.ShapeDtypeStruct((1, 1), jnp.float32),
        ],
    )(xp, W, b, att)


def _segsum_body(src_h, dst_h, a2_h, zero_h, s2_h, a2_v, srcb, dstb, pb, s_sh):
    cid = lax.axis_index("c")
    sid = lax.axis_index("s")
    wid = cid * NS + sid
    pltpu.sync_copy(a2_h, a2_v)

    @pl.when(sid == 0)
    def _():
        pltpu.sync_copy(zero_h, s_sh)

    plsc.subcore_barrier()

    base = wid * EPT

    def chunk(ci, carry):
        off = base + ci * CA
        pltpu.sync_copy(src_h.at[pl.ds(off, CA)], srcb)
        pltpu.sync_copy(dst_h.at[pl.ds(off, CA)], dstb)

        def vec(i, c2):
            s = srcb[pl.ds(i * 16, 16)]
            av = plsc.load_gather(a2_v, [s])
            pb[pl.ds(i * 16, 16)] = jnp.exp(av)
            return c2

        lax.fori_loop(0, CA // 16, vec, 0)
        pltpu.sync_copy(pb, s_sh.at[dstb], add=True)
        return carry

    lax.fori_loop(0, EPT // CA, chunk, 0)
    plsc.subcore_barrier()

    @pl.when(sid == 0)
    def _():
        pltpu.sync_copy(s_sh, s2_h.at[pl.ds(cid * NP, NP)])


_segsum = functools.partial(
    pl.kernel,
    out_type=jax.ShapeDtypeStruct((2 * NP,), jnp.float32),
    mesh=_mesh,
    scratch_types=[
        pltpu.VMEM((N,), jnp.float32),
        pltpu.VMEM((CA,), jnp.int32),
        pltpu.VMEM((CA,), jnp.int32),
        pltpu.VMEM((CA,), jnp.float32),
        pltpu.VMEM_SHARED((NP,), jnp.float32),
    ],
    compiler_params=_sc_params,
)(_segsum_body)


def _edgew_body(src_h, dst_h, a2_h, s2_h, w_h, a2_v, sinv, t0, t1, srcb, dstb, wb):
    cid = lax.axis_index("c")
    sid = lax.axis_index("s")
    wid = cid * NS + sid
    pltpu.sync_copy(a2_h, a2_v)

    def stage(k, carry):
        pltpu.sync_copy(s2_h.at[pl.ds(k * QS, QS)], t0)
        pltpu.sync_copy(s2_h.at[pl.ds(NP + k * QS, QS)], t1)

        def vec(i, c2):
            s = t0[pl.ds(i * 16, 16)] + t1[pl.ds(i * 16, 16)]
            sinv[pl.ds(k * QS + i * 16, 16)] = 1.0 / (s + 1e-16)
            return c2

        lax.fori_loop(0, QS // 16, vec, 0)
        return carry

    lax.fori_loop(0, N // QS, stage, 0)

    base = wid * EPT

    def chunk(ci, carry):
        off = base + ci * CA
        pltpu.sync_copy(src_h.at[pl.ds(off, CA)], srcb)
        pltpu.sync_copy(dst_h.at[pl.ds(off, CA)], dstb)

        def vec(i, c2):
            s = srcb[pl.ds(i * 16, 16)]
            d = dstb[pl.ds(i * 16, 16)]
            p = jnp.exp(plsc.load_gather(a2_v, [s]))
            iv = plsc.load_gather(sinv, [d])
            wb[pl.ds(i * 16, 16)] = p * iv
            return c2

        lax.fori_loop(0, CA // 16, vec, 0)
        pltpu.sync_copy(wb, w_h.at[pl.ds(off, CA)])
        return carry

    lax.fori_loop(0, EPT // CA, chunk, 0)


_edgew = functools.partial(
    pl.kernel,
    out_type=jax.ShapeDtypeStruct((E,), jnp.float32),
    mesh=_mesh,
    scratch_types=[
        pltpu.VMEM((N,), jnp.float32),
        pltpu.VMEM((N,), jnp.float32),
        pltpu.VMEM((QS,), jnp.float32),
        pltpu.VMEM((QS,), jnp.float32),
        pltpu.VMEM((CA,), jnp.int32),
        pltpu.VMEM((CA,), jnp.int32),
        pltpu.VMEM((CA,), jnp.float32),
    ],
    compiler_params=_sc_params,
)(_edgew_body)


def _minagg_body(src_h, dst_h, w_h, ypk_h, outPk_h,
                 ybuf, outb, dupchk,
                 sb0, db0, wb0, sb1, db1, wb1, sem0, sem1):
    cid = lax.axis_index("c")
    sid = lax.axis_index("s")

    pltpu.sync_copy(ypk_h.at[pl.ds(sid * 4 * NP, 4 * NP)], ybuf)

    inf16 = jnp.full((16,), 0x7F807F80, jnp.int32)

    def init(i, carry):
        outb[pl.ds(i * 16, 16)] = inf16
        return carry

    lax.fori_loop(0, (4 * NP) // 16, init, 0)

    sbufs = (sb0, sb1)
    dbufs = (db0, db1)
    wbufs = (wb0, wb1)
    sems = (sem0, sem1)
    EH = E // 2
    ebase = cid * EH
    nchunks = EH // CB
    nv = CB // 16
    iota16 = lax.iota(jnp.int32, 16)

    def start(ci, b):
        off = ebase + ci * CB
        pltpu.async_copy(src_h.at[pl.ds(off, CB)], sbufs[b], sems[b])
        pltpu.async_copy(dst_h.at[pl.ds(off, CB)], dbufs[b], sems[b])
        pltpu.async_copy(w_h.at[pl.ds(off, CB)], wbufs[b], sems[b])

    def wait(ci, b):
        off = ebase + ci * CB
        pltpu.make_async_copy(src_h.at[pl.ds(off, CB)], sbufs[b], sems[b]).wait()
        pltpu.make_async_copy(dst_h.at[pl.ds(off, CB)], dbufs[b], sems[b]).wait()
        pltpu.make_async_copy(w_h.at[pl.ds(off, CB)], wbufs[b], sems[b]).wait()

    for b in (0, 1):
        start(b, b)

    iota16b = iota16 + 16
    m16 = jnp.int32(-65536)

    def load_group(sb, db, wb, i):
        s = sb[pl.ds(i * 16, 16)]
        d = db[pl.ds(i * 16, 16)]
        w = wb[pl.ds(i * 16, 16)]
        vs = []
        for k in range(4):
            yk = plsc.load_gather(ybuf, [s + k * NP])
            flo = lax.bitcast_convert_type(lax.shift_left(yk, 16), jnp.float32)
            fhi = lax.bitcast_convert_type(lax.bitwise_and(yk, m16), jnp.float32)
            vp = plsc.pack(w * flo, w * fhi, format=plsc.PackFormat.INTERLEAVED)
            vs.append(plsc.bitcast(vp, jnp.int32))
        return d, vs

    def _bmin(a32, b32):
        a = plsc.bitcast(a32, jnp.bfloat16)
        b = plsc.bitcast(b32, jnp.bfloat16)
        return plsc.bitcast(jnp.minimum(a, b), jnp.int32)

    def rmw_fast(d, vs):
        for k in range(4):
            dd = d + k * NP
            c = plsc.load_gather(outb, [dd])
            plsc.store_scatter(outb, [dd], _bmin(c, vs[k]))

    def rmw_retry(d, vs):
        def cond(m):
            return plsc.all_reduce_population_count(m)[0] > 0

        def body(m):
            lost = jnp.zeros((16,), jnp.bool_)
            for k in range(4):
                dd = d + k * NP
                c = plsc.load_gather(outb, [dd], mask=m)
                nvv = _bmin(c, vs[k])
                plsc.store_scatter(outb, [dd], nvv, mask=m)
                chk = plsc.load_gather(outb, [dd], mask=m)
                lo_c = lax.bitcast_convert_type(lax.shift_left(chk, 16), jnp.float32)
                hi_c = lax.bitcast_convert_type(lax.bitwise_and(chk, m16), jnp.float32)
                lo_n = lax.bitcast_convert_type(lax.shift_left(nvv, 16), jnp.float32)
                hi_n = lax.bitcast_convert_type(lax.bitwise_and(nvv, m16), jnp.float32)
                bad = jnp.logical_or(lo_c > lo_n, hi_c > hi_n)
                lost = jnp.logical_or(lost, jnp.logical_and(m, bad))
            return lost

        lax.while_loop(cond, body, jnp.full((16,), True, jnp.bool_))

    def process(b):
        sb, db, wb = sbufs[b], dbufs[b], wbufs[b]

        def vec(i, carry):
            dA, vA = load_group(sb, db, wb, 2 * i)
            dB, vB = load_group(sb, db, wb, 2 * i + 1)
            plsc.store_scatter(dupchk, [dA], iota16)
            plsc.store_scatter(dupchk, [dB], iota16b)
            rdA = plsc.load_gather(dupchk, [dA])
            rdB = plsc.load_gather(dupchk, [dB])
            bad = jnp.logical_or(rdA != iota16, rdB != iota16b)
            nbad = plsc.all_reduce_population_count(bad)[0]

            def fast(_):
                rmw_fast(dA, vA)
                rmw_fast(dB, vB)
                return 0

            def slow(_):
                rmw_retry(dA, vA)
                rmw_retry(dB, vB)
                return 0

            lax.cond(nbad == 0, fast, slow, 0)
            return carry

        lax.fori_loop(0, nv // 2, vec, 0)

    def pair(cj, carry):
        for b in (0, 1):
            ci = cj * 2 + b
            wait(ci, b)
            process(b)
            nci = ci + 2

            @pl.when(nci < nchunks)
            def _():
                start(nci, b)

        return carry

    lax.fori_loop(0, nchunks // 2, pair, 0)
    pltpu.sync_copy(outb, outPk_h.at[pl.ds((cid * NS + sid) * 4 * NP, 4 * NP)])


_minagg = functools.partial(
    pl.kernel,
    out_type=jax.ShapeDtypeStruct((2 * (D // 2) * NP,), jnp.int32),
    mesh=_mesh,
    scratch_types=[
        pltpu.VMEM((4 * NP,), jnp.int32),
        pltpu.VMEM((4 * NP,), jnp.int32),
        pltpu.VMEM((N,), jnp.int32),
        pltpu.VMEM((CB,), jnp.int32),
        pltpu.VMEM((CB,), jnp.int32),
        pltpu.VMEM((CB,), jnp.float32),
        pltpu.VMEM((CB,), jnp.int32),
        pltpu.VMEM((CB,), jnp.int32),
        pltpu.VMEM((CB,), jnp.float32),
        pltpu.SemaphoreType.DMA,
        pltpu.SemaphoreType.DMA,
    ],
    compiler_params=_sc_params,
)(_minagg_body)


def _epi_body(p0_ref, p1_ref, s2_ref, x_ref, o_ref):
    u0 = p0_ref[...]
    u1 = p1_ref[...]
    m16 = jnp.int32(-65536)
    lo0 = lax.bitcast_convert_type(lax.shift_left(u0, 16), jnp.float32)
    hi0 = lax.bitcast_convert_type(lax.bitwise_and(u0, m16), jnp.float32)
    lo1 = lax.bitcast_convert_type(lax.shift_left(u1, 16), jnp.float32)
    hi1 = lax.bitcast_convert_type(lax.bitwise_and(u1, m16), jnp.float32)
    agg_lo = jnp.minimum(lo0, lo1)
    agg_hi = jnp.minimum(hi0, hi1)
    s = s2_ref[0:1, :] + s2_ref[1:2, :]
    agg_lo = jnp.where(s > 0, agg_lo, 0.0)
    agg_hi = jnp.where(s > 0, agg_hi, 0.0)
    x = x_ref[...]
    o_ref[:, 0:64] = agg_lo.T + x[:, 0:64]
    o_ref[:, 64:128] = agg_hi.T + x[:, 64:128]


def _epilogue(outPk0, outPk1, s2, xp):
    nb = NP // 1024
    return pl.pallas_call(
        _epi_body,
        grid=(nb,),
        in_specs=[
            pl.BlockSpec((D // 2, 1024), lambda j: (0, j)),
            pl.BlockSpec((D // 2, 1024), lambda j: (0, j)),
            pl.BlockSpec((2, 1024), lambda j: (0, j)),
            pl.BlockSpec((1024, D), lambda j: (j, 0)),
        ],
        out_specs=pl.BlockSpec((1024, D), lambda j: (j, 0)),
        out_shape=jax.ShapeDtypeStruct((NP, D), jnp.float32),
    )(outPk0, outPk1, s2, xp)


def kernel(x, edge_index, W_msg, b_msg, att_msg):
    src = edge_index[0]
    dst = edge_index[1]
    xp = jnp.pad(x, ((0, NP - N), (0, 0)))
    yT, a, m = _dense(xp, W_msg, b_msg.reshape(1, D), att_msg.reshape(D, 1))
    a2 = a[0, :N] - m[0, 0]
    zeros = jnp.zeros((NP,), jnp.float32)
    s2 = _segsum(src, dst, a2, zeros)
    w = _edgew(src, dst, a2, s2)
    yb16 = lax.bitcast_convert_type(yT.astype(jnp.bfloat16), jnp.uint16)
    ypk = (yb16[: D // 2, :].astype(jnp.uint32)
           | (yb16[D // 2:, :].astype(jnp.uint32) << 16))
    ypk = lax.bitcast_convert_type(ypk, jnp.int32).reshape(D // 2 * NP)
    outPk = _minagg(src, dst, w, ypk).reshape(2, D // 2, NP)
    out = _epilogue(outPk[0], outPk[1], s2.reshape(2, NP), xp)
    return out[:N]


# speedup vs baseline: 13.4561x; 1.0512x over previous
import functools

import jax
import jax.numpy as jnp
from jax import lax
from jax.experimental import pallas as pl
from jax.experimental.pallas import tpu as pltpu
from jax.experimental.pallas import tpu_sc as plsc

N = 10000
NP = 10240
E = 320000
D = 128

NC = 2
NS = 16
NW = NC * NS

EPT = E // NW
CA = 2000
CB = 3200
QS = 2000
FPT = D // NW

_mesh = plsc.VectorSubcoreMesh(core_axis_name="c", subcore_axis_name="s")
_sc_params = pltpu.CompilerParams(needs_layout_passes=False)


def _dense_body(x_ref, W_ref, b_ref, att_ref, yT_ref, a_ref, m_ref):
    j = pl.program_id(0)
    y = jnp.dot(x_ref[...], W_ref[...], preferred_element_type=jnp.float32)
    y = y + b_ref[...]
    yT = y.T
    yT_ref[...] = yT
    av = jnp.sum(yT * att_ref[...], axis=0, keepdims=True)
    av = jnp.where(av > 0, av, 0.2 * av)
    a_ref[...] = av
    bm = jnp.max(av)

    @pl.when(j == 0)
    def _():
        m_ref[0, 0] = bm

    @pl.when(j > 0)
    def _():
        m_ref[0, 0] = jnp.maximum(m_ref[0, 0], bm)


def _dense(xp, W, b, att):
    nb = NP // 1024
    return pl.pallas_call(
        _dense_body,
        grid=(nb,),
        in_specs=[
            pl.BlockSpec((1024, D), lambda j: (j, 0)),
            pl.BlockSpec((D, D), lambda j: (0, 0)),
            pl.BlockSpec((1, D), lambda j: (0, 0)),
            pl.BlockSpec((D, 1), lambda j: (0, 0)),
        ],
        out_specs=[
            pl.BlockSpec((D, 1024), lambda j: (0, j)),
            pl.BlockSpec((1, 1024), lambda j: (0, j)),
            pl.BlockSpec(memory_space=pltpu.SMEM),
        ],
        out_shape=[
            jax.ShapeDtypeStruct((D, NP), jnp.float32),
            jax.ShapeDtypeStruct((1, NP), jnp.float32),
            jax.ShapeDtypeStruct((1, 1), jnp.float32),
        ],
    )(xp, W, b, att)


def _segsum_body(src_h, dst_h, a2_h, zero_h, s2_h, a2_v, srcb, dstb, pb, s_sh):
    cid = lax.axis_index("c")
    sid = lax.axis_index("s")
    wid = cid * NS + sid
    pltpu.sync_copy(a2_h, a2_v)

    @pl.when(sid == 0)
    def _():
        pltpu.sync_copy(zero_h, s_sh)

    plsc.subcore_barrier()

    base = wid * EPT

    def chunk(ci, carry):
        off = base + ci * CA
        pltpu.sync_copy(src_h.at[pl.ds(off, CA)], srcb)
        pltpu.sync_copy(dst_h.at[pl.ds(off, CA)], dstb)

        def vec(i, c2):
            s = srcb[pl.ds(i * 16, 16)]
            av = plsc.load_gather(a2_v, [s])
            pb[pl.ds(i * 16, 16)] = jnp.exp(av)
            return c2

        lax.fori_loop(0, CA // 16, vec, 0)
        pltpu.sync_copy(pb, s_sh.at[dstb], add=True)
        return carry

    lax.fori_loop(0, EPT // CA, chunk, 0)
    plsc.subcore_barrier()

    @pl.when(sid == 0)
    def _():
        pltpu.sync_copy(s_sh, s2_h.at[pl.ds(cid * NP, NP)])


_segsum = functools.partial(
    pl.kernel,
    out_type=jax.ShapeDtypeStruct((2 * NP,), jnp.float32),
    mesh=_mesh,
    scratch_types=[
        pltpu.VMEM((N,), jnp.float32),
        pltpu.VMEM((CA,), jnp.int32),
        pltpu.VMEM((CA,), jnp.int32),
        pltpu.VMEM((CA,), jnp.float32),
        pltpu.VMEM_SHARED((NP,), jnp.float32),
    ],
    compiler_params=_sc_params,
)(_segsum_body)


def _edgew_body(src_h, dst_h, a2_h, s2_h, w_h, a2_v, sinv, t0, t1, srcb, dstb, wb):
    cid = lax.axis_index("c")
    sid = lax.axis_index("s")
    wid = cid * NS + sid
    pltpu.sync_copy(a2_h, a2_v)

    def stage(k, carry):
        pltpu.sync_copy(s2_h.at[pl.ds(k * QS, QS)], t0)
        pltpu.sync_copy(s2_h.at[pl.ds(NP + k * QS, QS)], t1)

        def vec(i, c2):
            s = t0[pl.ds(i * 16, 16)] + t1[pl.ds(i * 16, 16)]
            sinv[pl.ds(k * QS + i * 16, 16)] = 1.0 / (s + 1e-16)
            return c2

        lax.fori_loop(0, QS // 16, vec, 0)
        return carry

    lax.fori_loop(0, N // QS, stage, 0)

    base = wid * EPT

    def chunk(ci, carry):
        off = base + ci * CA
        pltpu.sync_copy(src_h.at[pl.ds(off, CA)], srcb)
        pltpu.sync_copy(dst_h.at[pl.ds(off, CA)], dstb)

        def vec(i, c2):
            s = srcb[pl.ds(i * 16, 16)]
            d = dstb[pl.ds(i * 16, 16)]
            p = jnp.exp(plsc.load_gather(a2_v, [s]))
            iv = plsc.load_gather(sinv, [d])
            wb[pl.ds(i * 16, 16)] = p * iv
            return c2

        lax.fori_loop(0, CA // 16, vec, 0)
        pltpu.sync_copy(wb, w_h.at[pl.ds(off, CA)])
        return carry

    lax.fori_loop(0, EPT // CA, chunk, 0)


_edgew = functools.partial(
    pl.kernel,
    out_type=jax.ShapeDtypeStruct((E,), jnp.float32),
    mesh=_mesh,
    scratch_types=[
        pltpu.VMEM((N,), jnp.float32),
        pltpu.VMEM((N,), jnp.float32),
        pltpu.VMEM((QS,), jnp.float32),
        pltpu.VMEM((QS,), jnp.float32),
        pltpu.VMEM((CA,), jnp.int32),
        pltpu.VMEM((CA,), jnp.int32),
        pltpu.VMEM((CA,), jnp.float32),
    ],
    compiler_params=_sc_params,
)(_edgew_body)


def _minagg_body(src_h, dst_h, w_h, ypk_h, outPk_h,
                 ybuf, outb, dupchk,
                 sb0, db0, wb0, sb1, db1, wb1, sem0, sem1):
    cid = lax.axis_index("c")
    sid = lax.axis_index("s")

    pltpu.sync_copy(ypk_h.at[pl.ds(sid * 4 * NP, 4 * NP)], ybuf)

    inf16 = jnp.full((16,), 0x7F807F80, jnp.int32)

    def init(i, carry):
        outb[pl.ds(i * 16, 16)] = inf16
        return carry

    lax.fori_loop(0, (4 * NP) // 16, init, 0)

    sbufs = (sb0, sb1)
    dbufs = (db0, db1)
    wbufs = (wb0, wb1)
    sems = (sem0, sem1)
    EH = E // 2
    ebase = cid * EH
    nchunks = EH // CB
    nv = CB // 16
    iota16 = lax.iota(jnp.int32, 16)

    def start(ci, b):
        off = ebase + ci * CB
        pltpu.async_copy(src_h.at[pl.ds(off, CB)], sbufs[b], sems[b])
        pltpu.async_copy(dst_h.at[pl.ds(off, CB)], dbufs[b], sems[b])
        pltpu.async_copy(w_h.at[pl.ds(off, CB)], wbufs[b], sems[b])

    def wait(ci, b):
        off = ebase + ci * CB
        pltpu.make_async_copy(src_h.at[pl.ds(off, CB)], sbufs[b], sems[b]).wait()
        pltpu.make_async_copy(dst_h.at[pl.ds(off, CB)], dbufs[b], sems[b]).wait()
        pltpu.make_async_copy(w_h.at[pl.ds(off, CB)], wbufs[b], sems[b]).wait()

    for b in (0, 1):
        start(b, b)

    m16 = jnp.int32(-65536)

    def load_group(sb, db, wb, i):
        s = sb[pl.ds(i * 16, 16)]
        d = db[pl.ds(i * 16, 16)]
        w = wb[pl.ds(i * 16, 16)]
        vs = []
        for k in range(4):
            yk = plsc.load_gather(ybuf, [s + k * NP])
            flo = lax.bitcast_convert_type(lax.shift_left(yk, 16), jnp.float32)
            fhi = lax.bitcast_convert_type(lax.bitwise_and(yk, m16), jnp.float32)
            vp = plsc.pack(w * flo, w * fhi, format=plsc.PackFormat.INTERLEAVED)
            vs.append(plsc.bitcast(vp, jnp.int32))
        return d, vs

    def _bmin(a32, b32):
        a = plsc.bitcast(a32, jnp.bfloat16)
        b = plsc.bitcast(b32, jnp.bfloat16)
        return plsc.bitcast(jnp.minimum(a, b), jnp.int32)

    def rmw_fast(d, vs):
        for k in range(4):
            dd = d + k * NP
            c = plsc.load_gather(outb, [dd])
            plsc.store_scatter(outb, [dd], _bmin(c, vs[k]))

    def rmw_retry(d, vs):
        def cond(m):
            return plsc.all_reduce_population_count(m)[0] > 0

        def body(m):
            lost = jnp.zeros((16,), jnp.bool_)
            for k in range(4):
                dd = d + k * NP
                c = plsc.load_gather(outb, [dd], mask=m)
                nvv = _bmin(c, vs[k])
                plsc.store_scatter(outb, [dd], nvv, mask=m)
                chk = plsc.load_gather(outb, [dd], mask=m)
                lo_c = lax.bitcast_convert_type(lax.shift_left(chk, 16), jnp.float32)
                hi_c = lax.bitcast_convert_type(lax.bitwise_and(chk, m16), jnp.float32)
                lo_n = lax.bitcast_convert_type(lax.shift_left(nvv, 16), jnp.float32)
                hi_n = lax.bitcast_convert_type(lax.bitwise_and(nvv, m16), jnp.float32)
                bad = jnp.logical_or(lo_c > lo_n, hi_c > hi_n)
                lost = jnp.logical_or(lost, jnp.logical_and(m, bad))
            return lost

        lax.while_loop(cond, body, jnp.full((16,), True, jnp.bool_))

    def process(b):
        sb, db, wb = sbufs[b], dbufs[b], wbufs[b]

        def vec(i, carry):
            groups = [load_group(sb, db, wb, 4 * i + g) for g in range(4)]
            for g, (dg, _) in enumerate(groups):
                plsc.store_scatter(dupchk, [dg], iota16 + 16 * g)
            bad = jnp.zeros((16,), jnp.bool_)
            for g, (dg, _) in enumerate(groups):
                rd = plsc.load_gather(dupchk, [dg])
                bad = jnp.logical_or(bad, rd != iota16 + 16 * g)
            nbad = plsc.all_reduce_population_count(bad)[0]

            def fast(_):
                for dg, vg in groups:
                    rmw_fast(dg, vg)
                return 0

            def slow(_):
                for dg, vg in groups:
                    rmw_retry(dg, vg)
                return 0

            lax.cond(nbad == 0, fast, slow, 0)
            return carry

        lax.fori_loop(0, nv // 4, vec, 0)

    def pair(cj, carry):
        for b in (0, 1):
            ci = cj * 2 + b
            wait(ci, b)
            process(b)
            nci = ci + 2

            @pl.when(nci < nchunks)
            def _():
                start(nci, b)

        return carry

    lax.fori_loop(0, nchunks // 2, pair, 0)
    pltpu.sync_copy(outb, outPk_h.at[pl.ds((cid * NS + sid) * 4 * NP, 4 * NP)])


_minagg = functools.partial(
    pl.kernel,
    out_type=jax.ShapeDtypeStruct((2 * (D // 2) * NP,), jnp.int32),
    mesh=_mesh,
    scratch_types=[
        pltpu.VMEM((4 * NP,), jnp.int32),
        pltpu.VMEM((4 * NP,), jnp.int32),
        pltpu.VMEM((N,), jnp.int32),
        pltpu.VMEM((CB,), jnp.int32),
        pltpu.VMEM((CB,), jnp.int32),
        pltpu.VMEM((CB,), jnp.float32),
        pltpu.VMEM((CB,), jnp.int32),
        pltpu.VMEM((CB,), jnp.int32),
        pltpu.VMEM((CB,), jnp.float32),
        pltpu.SemaphoreType.DMA,
        pltpu.SemaphoreType.DMA,
    ],
    compiler_params=_sc_params,
)(_minagg_body)


def _epi_body(p0_ref, p1_ref, s2_ref, x_ref, o_ref):
    u0 = p0_ref[...]
    u1 = p1_ref[...]
    m16 = jnp.int32(-65536)
    lo0 = lax.bitcast_convert_type(lax.shift_left(u0, 16), jnp.float32)
    hi0 = lax.bitcast_convert_type(lax.bitwise_and(u0, m16), jnp.float32)
    lo1 = lax.bitcast_convert_type(lax.shift_left(u1, 16), jnp.float32)
    hi1 = lax.bitcast_convert_type(lax.bitwise_and(u1, m16), jnp.float32)
    agg_lo = jnp.minimum(lo0, lo1)
    agg_hi = jnp.minimum(hi0, hi1)
    s = s2_ref[0:1, :] + s2_ref[1:2, :]
    agg_lo = jnp.where(s > 0, agg_lo, 0.0)
    agg_hi = jnp.where(s > 0, agg_hi, 0.0)
    x = x_ref[...]
    o_ref[:, 0:64] = agg_lo.T + x[:, 0:64]
    o_ref[:, 64:128] = agg_hi.T + x[:, 64:128]


def _epilogue(outPk0, outPk1, s2, xp):
    nb = NP // 1024
    return pl.pallas_call(
        _epi_body,
        grid=(nb,),
        in_specs=[
            pl.BlockSpec((D // 2, 1024), lambda j: (0, j)),
            pl.BlockSpec((D // 2, 1024), lambda j: (0, j)),
            pl.BlockSpec((2, 1024), lambda j: (0, j)),
            pl.BlockSpec((1024, D), lambda j: (j, 0)),
        ],
        out_specs=pl.BlockSpec((1024, D), lambda j: (j, 0)),
        out_shape=jax.ShapeDtypeStruct((NP, D), jnp.float32),
    )(outPk0, outPk1, s2, xp)


def kernel(x, edge_index, W_msg, b_msg, att_msg):
    src = edge_index[0]
    dst = edge_index[1]
    xp = jnp.pad(x, ((0, NP - N), (0, 0)))
    yT, a, m = _dense(xp, W_msg, b_msg.reshape(1, D), att_msg.reshape(D, 1))
    a2 = a[0, :N] - m[0, 0]
    zeros = jnp.zeros((NP,), jnp.float32)
    s2 = _segsum(src, dst, a2, zeros)
    w = _edgew(src, dst, a2, s2)
    yb16 = lax.bitcast_convert_type(yT.astype(jnp.bfloat16), jnp.uint16)
    ypk = (yb16[: D // 2, :].astype(jnp.uint32)
           | (yb16[D // 2:, :].astype(jnp.uint32) << 16))
    ypk = lax.bitcast_convert_type(ypk, jnp.int32).reshape(D // 2 * NP)
    outPk = _minagg(src, dst, w, ypk).reshape(2, D // 2, NP)
    out = _epilogue(outPk[0], outPk[1], s2.reshape(2, NP), xp)
    return out[:N]
